# baseline (speedup 1.0000x reference)
import jax
import jax.numpy as jnp
from jax import lax
from jax.experimental import pallas as pl
from jax.experimental.pallas import tpu as pltpu
from jax.experimental.pallas import tpu_sc as plsc

_D = 2048
_BR = 1024
_L = 16
_NCHUNK = _D // _L


def _sc_mask_body(sel_ref, k_ref, kvec_ref, kk_ref, sel_v, k_v, kv_v, key_v, out_v):
    cid = lax.axis_index("c")
    sid = lax.axis_index("s")

    @pl.when(jnp.logical_and(cid == 0, sid == 0))
    def _():
        pltpu.sync_copy(kvec_ref, kv_v)
        pltpu.sync_copy(kv_v, kk_ref)


def _sc_mask(sel_splat, k_splat, kvec):
    return pl.kernel(
        _sc_mask_body,
        out_type=jax.ShapeDtypeStruct((_D,), jnp.float32),
        mesh=plsc.VectorSubcoreMesh(core_axis_name="c", subcore_axis_name="s"),
        compiler_params=pltpu.CompilerParams(needs_layout_passes=False),
        scratch_types=[
            pltpu.VMEM((_L,), jnp.int32),
            pltpu.VMEM((_L,), jnp.int32),
            pltpu.VMEM((_D,), jnp.float32),
            pltpu.VMEM((_D,), jnp.uint32),
            pltpu.VMEM((_D,), jnp.float32),
        ],
    )(sel_splat, k_splat, kvec)


def _scale_body(kk_ref, x_ref, out_ref):
    out_ref[...] = x_ref[...] * kk_ref[...]


def kernel(x, kernel, selection, k):
    n_rows = x.shape[0]
    sel_splat = jnp.full((_L,), jnp.asarray(selection, jnp.int32))
    k_splat = jnp.full((_L,), jnp.asarray(k, jnp.int32))

    kk = _sc_mask(sel_splat, k_splat, kernel).reshape(1, _D)

    return pl.pallas_call(
        _scale_body,
        grid=(n_rows // _BR,),
        in_specs=[
            pl.BlockSpec((1, _D), lambda i: (0, 0)),
            pl.BlockSpec((_BR, _D), lambda i: (i, 0)),
        ],
        out_specs=pl.BlockSpec((_BR, _D), lambda i: (i, 0)),
        out_shape=jax.ShapeDtypeStruct(x.shape, x.dtype),
    )(kk, x)


# speedup vs baseline: 1.2003x; 1.2003x over previous
import jax
import jax.numpy as jnp
from jax import lax
from jax.experimental import pallas as pl
from jax.experimental.pallas import tpu as pltpu

_D = 2048
_BR = 1024


def _monotone_key(v):
    b = lax.bitcast_convert_type(v, jnp.int32)
    u = lax.bitcast_convert_type(v, jnp.uint32)
    return jnp.where(b < 0, ~u, u | jnp.uint32(0x80000000))


def _body(sel_ref, k_ref, kvec8_ref, kvec_ref, x_ref, out_ref, kk_ref):
    @pl.when(pl.program_id(0) == 0)
    def _prologue():
        key8 = _monotone_key(kvec8_ref[...])
        k = k_ref[0]
        skey8 = lax.bitcast_convert_type(
            key8 ^ jnp.uint32(0x80000000), jnp.int32)
        min_s = jnp.min(skey8)
        max_s = jnp.max(skey8)
        min_u = lax.bitcast_convert_type(min_s, jnp.uint32) ^ jnp.uint32(0x80000000)
        diff_i = min_s ^ max_s
        est = (lax.bitcast_convert_type(
            diff_i.astype(jnp.float32), jnp.int32) >> 23) - 127
        start = jnp.where(diff_i < 0, jnp.int32(31),
                          jnp.clip(est, 0, 31)).astype(jnp.uint32)
        acc0 = min_u & ~((jnp.uint32(2) << start) - jnp.uint32(1))

        def step(j, acc):
            bit = jnp.uint32(1) << (start - j.astype(jnp.uint32))
            cand = acc | bit
            cnt = jnp.sum((key8 >= cand).astype(jnp.int32))
            return jnp.where(cnt >= k, cand, acc)

        acc = lax.fori_loop(0, start.astype(jnp.int32) + 1, step, acc0)
        kv = kvec_ref[...]
        masked = jnp.where(_monotone_key(kv) < acc, jnp.float32(0.0), kv)
        kk_ref[...] = jnp.where(sel_ref[0] != 0, masked, kv)

    out_ref[...] = x_ref[...] * kk_ref[...]


def kernel(x, kernel, selection, k):
    n_rows = x.shape[0]
    grid = (n_rows // _BR,)
    sel_arr = jnp.asarray(selection, jnp.int32).reshape(1)
    k_arr = jnp.asarray(k, jnp.int32).reshape(1)
    kvec8 = kernel.reshape(8, _D // 8)
    kvec = kernel.reshape(1, _D)

    return pl.pallas_call(
        _body,
        grid_spec=pltpu.PrefetchScalarGridSpec(
            num_scalar_prefetch=2,
            grid=grid,
            in_specs=[
                pl.BlockSpec((8, _D // 8), lambda i, *_: (0, 0)),
                pl.BlockSpec((1, _D), lambda i, *_: (0, 0)),
                pl.BlockSpec((_BR, _D), lambda i, *_: (i, 0)),
            ],
            out_specs=pl.BlockSpec((_BR, _D), lambda i, *_: (i, 0)),
            scratch_shapes=[pltpu.VMEM((1, _D), jnp.float32)],
        ),
        out_shape=jax.ShapeDtypeStruct(x.shape, x.dtype),
    )(sel_arr, k_arr, kvec8, kvec, x)
